# Initial kernel scaffold; baseline (speedup 1.0000x reference)
import jax
import jax.numpy as jnp
from jax.experimental import pallas as pl


def kernel(dense_features, W, sparse_0, sparse_1, sparse_2, sparse_3, sparse_4, sparse_5, sparse_6, sparse_7, sparse_8, sparse_9, sparse_10, sparse_11, sparse_12, sparse_13, sparse_14, sparse_15, sparse_16, sparse_17, sparse_18, sparse_19, sparse_20, sparse_21, sparse_22, sparse_23, sparse_24, sparse_25):
    raise NotImplementedError("write your pallas kernel here")


# speedup vs baseline: 38.7316x; 38.7316x over previous
import functools

import jax
import jax.numpy as jnp
from jax import lax
from jax.experimental import pallas as pl
from jax.experimental.pallas import tpu as pltpu
from jax.experimental.pallas import tpu_sc as plsc

B = 4096
F = 26
V = 1000
D = 13
W_LEN = F * V + D
W_PAD = 26016

NC = 2
NS = 16
NW = NC * NS
BPW = B // NW
L = 16
G = BPW // L


def _wide_sc(idx_blocks, dense_blocks, w_flat):
    mesh = plsc.VectorSubcoreMesh(core_axis_name="c", subcore_axis_name="s")

    @functools.partial(
        pl.kernel,
        mesh=mesh,
        out_type=jax.ShapeDtypeStruct((B,), jnp.float32),
        compiler_params=pltpu.CompilerParams(needs_layout_passes=False),
        scratch_types=[
            pltpu.VMEM((F, BPW), jnp.int32),
            pltpu.VMEM((D, BPW), jnp.float32),
            pltpu.VMEM((W_PAD,), jnp.float32),
            pltpu.VMEM((BPW,), jnp.float32),
        ],
    )
    def body(idx_hbm, dense_hbm, w_hbm, out_hbm, idx_v, dense_v, w_v, out_v):
        wid = lax.axis_index("s") * NC + lax.axis_index("c")
        pltpu.sync_copy(w_hbm, w_v)
        pltpu.sync_copy(idx_hbm.at[wid], idx_v)
        pltpu.sync_copy(dense_hbm.at[wid], dense_v)
        w016 = w_v[pl.ds(0, L)]
        for g in range(G):
            sl = pl.ds(g * L, L)
            acc = dense_v[0, sl] * w016[0]
            for d in range(1, D):
                acc = acc + dense_v[d, sl] * w016[d]
            for i in range(F):
                gidx = idx_v[i, sl] + (D + i * V)
                acc = acc + plsc.load_gather(w_v, [gidx])
            out_v[sl] = acc
        pltpu.sync_copy(out_v, out_hbm.at[pl.ds(wid * BPW, BPW)])

    return body(idx_blocks, dense_blocks, w_flat)


def kernel(dense_features, W,
           sparse_0, sparse_1, sparse_2, sparse_3, sparse_4, sparse_5,
           sparse_6, sparse_7, sparse_8, sparse_9, sparse_10, sparse_11,
           sparse_12, sparse_13, sparse_14, sparse_15, sparse_16, sparse_17,
           sparse_18, sparse_19, sparse_20, sparse_21, sparse_22, sparse_23,
           sparse_24, sparse_25):
    sparse = [sparse_0, sparse_1, sparse_2, sparse_3, sparse_4, sparse_5,
              sparse_6, sparse_7, sparse_8, sparse_9, sparse_10, sparse_11,
              sparse_12, sparse_13, sparse_14, sparse_15, sparse_16,
              sparse_17, sparse_18, sparse_19, sparse_20, sparse_21,
              sparse_22, sparse_23, sparse_24, sparse_25]
    idx = jnp.stack(sparse, axis=0).reshape(F, NW, BPW).transpose(1, 0, 2)
    dense_blocks = dense_features.T.reshape(D, NW, BPW).transpose(1, 0, 2)
    w_flat = jnp.pad(W[:, 0], (0, W_PAD - W_LEN))
    out = _wide_sc(idx, dense_blocks, w_flat)
    return out[:, None]


# speedup vs baseline: 40.3547x; 1.0419x over previous
import functools

import jax
import jax.numpy as jnp
from jax import lax
from jax.experimental import pallas as pl
from jax.experimental.pallas import tpu as pltpu
from jax.experimental.pallas import tpu_sc as plsc

B = 4096
F = 26
V = 1000
D = 13
W_LEN = F * V + D
W_PAD = 26016

NC = 2
NS = 16
NW = NC * NS
BPW = B // NW
L = 16
G = BPW // L


def _wide_sc(idx_blocks, dense_blocks, w_flat):
    mesh = plsc.VectorSubcoreMesh(core_axis_name="c", subcore_axis_name="s")

    @functools.partial(
        pl.kernel,
        mesh=mesh,
        out_type=jax.ShapeDtypeStruct((B,), jnp.float32),
        compiler_params=pltpu.CompilerParams(needs_layout_passes=False),
        scratch_types=[
            pltpu.VMEM((F, BPW), jnp.int32),
            pltpu.VMEM((D, BPW), jnp.float32),
            pltpu.VMEM((W_PAD,), jnp.float32),
            pltpu.VMEM((BPW,), jnp.float32),
            pltpu.SemaphoreType.DMA,
        ],
    )
    def body(idx_hbm, dense_hbm, w_hbm, out_hbm, idx_v, dense_v, w_v, out_v,
             sem):
        wid = lax.axis_index("s") * NC + lax.axis_index("c")
        c1 = pltpu.async_copy(w_hbm, w_v, sem)
        c2 = pltpu.async_copy(idx_hbm.at[wid], idx_v, sem)
        c3 = pltpu.async_copy(dense_hbm.at[wid], dense_v, sem)
        c1.wait()
        c2.wait()
        c3.wait()
        w016 = w_v[pl.ds(0, L)]
        for g in range(G):
            sl = pl.ds(g * L, L)
            acc = dense_v[0, sl] * w016[0]
            for d in range(1, D):
                acc = acc + dense_v[d, sl] * w016[d]
            for i in range(F):
                gidx = idx_v[i, sl] + (D + i * V)
                acc = acc + plsc.load_gather(w_v, [gidx])
            out_v[sl] = acc
        pltpu.sync_copy(out_v, out_hbm.at[pl.ds(wid * BPW, BPW)])

    return body(idx_blocks, dense_blocks, w_flat)


def kernel(dense_features, W,
           sparse_0, sparse_1, sparse_2, sparse_3, sparse_4, sparse_5,
           sparse_6, sparse_7, sparse_8, sparse_9, sparse_10, sparse_11,
           sparse_12, sparse_13, sparse_14, sparse_15, sparse_16, sparse_17,
           sparse_18, sparse_19, sparse_20, sparse_21, sparse_22, sparse_23,
           sparse_24, sparse_25):
    sparse = [sparse_0, sparse_1, sparse_2, sparse_3, sparse_4, sparse_5,
              sparse_6, sparse_7, sparse_8, sparse_9, sparse_10, sparse_11,
              sparse_12, sparse_13, sparse_14, sparse_15, sparse_16,
              sparse_17, sparse_18, sparse_19, sparse_20, sparse_21,
              sparse_22, sparse_23, sparse_24, sparse_25]
    idx = jnp.stack(sparse, axis=0).reshape(F, NW, BPW).transpose(1, 0, 2)
    dense_blocks = dense_features.T.reshape(D, NW, BPW).transpose(1, 0, 2)
    w_flat = jnp.pad(W[:, 0], (0, W_PAD - W_LEN))
    out = _wide_sc(idx, dense_blocks, w_flat)
    return out[:, None]
